# baseline (speedup 1.0000x reference)
import functools

import jax
import jax.numpy as jnp
from jax import lax
from jax.experimental import pallas as pl
from jax.experimental.pallas import tpu as pltpu
from jax.experimental.pallas import tpu_sc as plsc

NC = 2
NS = 16
NW = NC * NS


def _mesh():
    return plsc.VectorSubcoreMesh(core_axis_name="c", subcore_axis_name="s")


NBUF = 4


@functools.lru_cache(maxsize=None)
def _make_agg(ch, n_p, d, serial_scatter=False):
    npt = n_p // NS
    nv = d // 16
    assert ch % NBUF == 0 and ch >= 2 * NBUF

    @functools.partial(
        pl.kernel,
        mesh=_mesh(),
        out_type=jax.ShapeDtypeStruct((NC, n_p, d), jnp.float32),
        compiler_params=pltpu.CompilerParams(use_tc_tiling_on_sc=False),
        scratch_types=[
            pltpu.VMEM((ch, 128), jnp.int32),
            pltpu.VMEM((ch, 128), jnp.int32),
            pltpu.VMEM((ch, 128), jnp.float32),
            [pltpu.VMEM((128, d), jnp.float32)] * NBUF,
            [pltpu.SemaphoreType.DMA] * NBUF,
            [pltpu.SemaphoreType.DMA] * NBUF,
            pltpu.VMEM_SHARED((n_p, d), jnp.float32),
        ],
    )
    def agg_kernel(table_hbm, src_hbm, dst_hbm, w_hbm, zeros_hbm, out_hbm,
                   src_v, dst_v, w_v, rows, gsem, ssem, acc_sh):
        cid = lax.axis_index("c")
        sid = lax.axis_index("s")
        wid = cid * NS + sid
        sl = pl.ds(sid * npt, npt)
        pltpu.sync_copy(zeros_hbm.at[sl], acc_sh.at[sl])
        pltpu.sync_copy(src_hbm.at[wid], src_v)
        pltpu.sync_copy(dst_hbm.at[wid], dst_v)
        pltpu.sync_copy(w_hbm.at[wid], w_v)
        plsc.subcore_barrier()

        def gather_start(j, s):
            pltpu.async_copy(table_hbm.at[src_v.at[j]], rows[s], gsem[s])

        def gather_wait(j, s):
            pltpu.make_async_copy(table_hbm.at[src_v.at[j]], rows[s],
                                  gsem[s]).wait()

        def scale(j, s):
            rv = rows[s]
            for g in range(8):
                wv = w_v[j, pl.ds(g * 16, 16)]
                base = g * 16
                for i in range(16):
                    ws = jnp.take_along_axis(
                        wv, jnp.full((16,), i, jnp.int32), axis=0)
                    for k in range(nv):
                        s2 = pl.ds(k * 16, 16)
                        rv[base + i, s2] = rv[base + i, s2] * ws

        def scatter_start(j, s):
            pltpu.async_copy(rows[s], acc_sh.at[dst_v.at[j]], ssem[s],
                             add=True)

        def scatter_wait(j, s):
            pltpu.make_async_copy(rows[s], acc_sh.at[dst_v.at[j]],
                                  ssem[s]).wait()

        for s in range(NBUF):
            gather_start(s, s)

        if serial_scatter:
            def body(t, c):
                j0 = t * NBUF
                for s in range(NBUF):
                    j = j0 + s
                    sp = (s - 1) % NBUF
                    gather_wait(j, s)
                    scale(j, s)

                    @pl.when(j > 0)
                    def _():
                        scatter_wait(j - 1, sp)

                        @pl.when(j - 1 + NBUF < ch)
                        def _():
                            gather_start(j - 1 + NBUF, sp)

                    scatter_start(j, s)
                return c

            lax.fori_loop(0, ch // NBUF, body, 0)
            scatter_wait(ch - 1, NBUF - 1)
        else:
            def body(t, c):
                j0 = t * NBUF
                for s in range(NBUF):
                    j = j0 + s
                    gather_wait(j, s)
                    scale(j, s)
                    scatter_start(j, s)
                    if s >= 1:
                        jp = j - 1
                        scatter_wait(jp, s - 1)

                        @pl.when(jp + NBUF < ch)
                        def _():
                            gather_start(jp + NBUF, s - 1)

                jl = j0 + NBUF - 1
                scatter_wait(jl, NBUF - 1)

                @pl.when(jl + NBUF < ch)
                def _():
                    gather_start(jl + NBUF, NBUF - 1)

                return c

            lax.fori_loop(0, ch // NBUF, body, 0)
        plsc.subcore_barrier()
        pltpu.sync_copy(acc_sh.at[sl], out_hbm.at[cid, sl])

    return agg_kernel


def _tc1_body(x_ref, w1_ref, dsum_ref, out_ref):
    dinv = lax.rsqrt(dsum_ref[...])
    h = jnp.dot(x_ref[...], w1_ref[...], preferred_element_type=jnp.float32)
    out_ref[...] = h * dinv


def _tc2_body(acc_ref, h1s_ref, dsum_ref, b1_ref, w2_ref, out_ref):
    dinv = lax.rsqrt(dsum_ref[...])
    a = acc_ref[...]
    pre = (a[0] + a[1] + h1s_ref[...]) * dinv + b1_ref[...]
    h2 = jnp.where(pre > 0, pre, 0.01 * pre)
    out_ref[...] = jnp.dot(h2, w2_ref[...],
                           preferred_element_type=jnp.float32) * dinv


def _tc3_body(acc_ref, gs_ref, dsum_ref, b2_ref, out_ref):
    dinv = lax.rsqrt(dsum_ref[...])
    a = acc_ref[...]
    logits = (a[0] + a[1] + gs_ref[...]) * dinv + b2_ref[...]
    col = lax.broadcasted_iota(jnp.int32, logits.shape, 1)
    logits = jnp.where(col < 4, logits, -1e30)
    m = jnp.max(logits, axis=1, keepdims=True)
    ex = jnp.exp(logits - m)
    out_ref[...] = ex / jnp.sum(ex, axis=1, keepdims=True)


def kernel(x_embeddings, edge_index, weights, W1, b1, W2, b2):
    n, d_in = x_embeddings.shape
    e = edge_index.shape[1]
    d_h = W1.shape[1]
    d_out = W2.shape[1]

    n_p = ((n + 1023) // 1024) * 1024
    ch = (e + NW * 128 - 1) // (NW * 128)
    ch = ((ch + NBUF - 1) // NBUF) * NBUF
    e_p = NW * ch * 128
    pad = e_p - e
    d_out_p = 16

    f32 = jnp.float32
    src = edge_index[0]
    dst = edge_index[1]
    pad_idx = (jnp.arange(pad, dtype=jnp.int32) * 97) % n
    src_w = jnp.concatenate([src, pad_idx]).reshape(NW, ch, 128)
    dst_w = jnp.concatenate([dst, pad_idx]).reshape(NW, ch, 128)
    w_flat = jnp.concatenate([weights.astype(f32), jnp.zeros((pad,), f32)])
    w_w = w_flat.reshape(NW, ch, 128)

    zeros_h = jnp.zeros((n_p, d_h), f32)
    zeros_o = jnp.zeros((n_p, d_out_p), f32)
    x_p = jnp.pad(x_embeddings.astype(f32), ((0, n_p - n), (0, 0)))
    w2_p = jnp.pad(W2.astype(f32), ((0, 0), (0, d_out_p - d_out)))
    b1_r = b1.astype(f32).reshape(1, d_h)
    b2_r = jnp.pad(b2.astype(f32), (0, d_out_p - d_out)).reshape(1, d_out_p)

    ones_o = jnp.ones((n_p, d_out_p), f32)
    deg = _make_agg(ch, n_p, d_out_p)(ones_o, src_w, dst_w, w_w, zeros_o)
    dsum = deg[0, :, :1] + deg[1, :, :1] + 1.0

    r = 1024
    grid = (n_p // r,)

    h1s = pl.pallas_call(
        _tc1_body,
        grid=grid,
        in_specs=[
            pl.BlockSpec((r, d_in), lambda i: (i, 0)),
            pl.BlockSpec((d_in, d_h), lambda i: (0, 0)),
            pl.BlockSpec((r, 1), lambda i: (i, 0)),
        ],
        out_specs=pl.BlockSpec((r, d_h), lambda i: (i, 0)),
        out_shape=jax.ShapeDtypeStruct((n_p, d_h), f32),
    )(x_p, W1.astype(f32), dsum)

    acc1 = _make_agg(ch, n_p, d_h)(h1s, src_w, dst_w, w_w, zeros_h)

    gs = pl.pallas_call(
        _tc2_body,
        grid=grid,
        in_specs=[
            pl.BlockSpec((NC, r, d_h), lambda i: (0, i, 0)),
            pl.BlockSpec((r, d_h), lambda i: (i, 0)),
            pl.BlockSpec((r, 1), lambda i: (i, 0)),
            pl.BlockSpec((1, d_h), lambda i: (0, 0)),
            pl.BlockSpec((d_h, d_out_p), lambda i: (0, 0)),
        ],
        out_specs=pl.BlockSpec((r, d_out_p), lambda i: (i, 0)),
        out_shape=jax.ShapeDtypeStruct((n_p, d_out_p), f32),
    )(acc1, h1s, dsum, b1_r, w2_p)

    acc2 = _make_agg(ch, n_p, d_out_p)(gs, src_w, dst_w, w_w, zeros_o)

    out = pl.pallas_call(
        _tc3_body,
        grid=grid,
        in_specs=[
            pl.BlockSpec((NC, r, d_out_p), lambda i: (0, i, 0)),
            pl.BlockSpec((r, d_out_p), lambda i: (i, 0)),
            pl.BlockSpec((r, 1), lambda i: (i, 0)),
            pl.BlockSpec((1, d_out_p), lambda i: (0, 0)),
        ],
        out_specs=pl.BlockSpec((r, d_out_p), lambda i: (i, 0)),
        out_shape=jax.ShapeDtypeStruct((n_p, d_out_p), f32),
    )(acc2, gs, dsum, b2_r)

    return out[:n, :d_out]


# speedup vs baseline: 1.0368x; 1.0368x over previous
import functools

import jax
import jax.numpy as jnp
from jax import lax
from jax.experimental import pallas as pl
from jax.experimental.pallas import tpu as pltpu
from jax.experimental.pallas import tpu_sc as plsc

NC = 2
NS = 16
NW = NC * NS


def _mesh():
    return plsc.VectorSubcoreMesh(core_axis_name="c", subcore_axis_name="s")


NBUF = 4


@functools.lru_cache(maxsize=None)
def _make_deg(ch, n_p):
    npt = n_p // NS
    d = 16

    @functools.partial(
        pl.kernel,
        mesh=_mesh(),
        out_type=jax.ShapeDtypeStruct((NC, n_p, d), jnp.float32),
        compiler_params=pltpu.CompilerParams(use_tc_tiling_on_sc=False),
        scratch_types=[
            pltpu.VMEM((ch, 128), jnp.int32),
            pltpu.VMEM((ch, 128), jnp.float32),
            [pltpu.VMEM((128, d), jnp.float32)] * NBUF,
            [pltpu.SemaphoreType.DMA] * NBUF,
            pltpu.VMEM_SHARED((n_p, d), jnp.float32),
        ],
    )
    def deg_kernel(dst_hbm, w_hbm, zb_hbm, out_hbm, dst_v, w_v, rows, ssem,
                   acc_sh):
        cid = lax.axis_index("c")
        sid = lax.axis_index("s")
        wid = cid * NS + sid
        sl = pl.ds(sid * npt, npt)
        for s in range(NBUF):
            pltpu.sync_copy(zb_hbm, rows[s])
        for b in range(npt // 128):
            pltpu.sync_copy(rows[0],
                            acc_sh.at[pl.ds(sid * npt + b * 128, 128)])
        pltpu.sync_copy(dst_hbm.at[wid], dst_v)
        pltpu.sync_copy(w_hbm.at[wid], w_v)
        plsc.subcore_barrier()

        def fill(j, s):
            rv = rows[s]
            for g in range(8):
                wv = w_v[j, pl.ds(g * 16, 16)]
                for i in range(16):
                    ws = jnp.take_along_axis(
                        wv, jnp.full((16,), i, jnp.int32), axis=0)
                    sl16 = pl.ds(0, 16)
                    old = rv[g * 16 + i, sl16]
                    rv[g * 16 + i, sl16] = old - old + ws

        def scatter_start(j, s):
            pltpu.async_copy(rows[s], acc_sh.at[dst_v.at[j]], ssem[s],
                             add=True)

        def scatter_wait(j, s):
            pltpu.make_async_copy(rows[s], acc_sh.at[dst_v.at[j]],
                                  ssem[s]).wait()

        def body(t, c):
            for s in range(NBUF):
                j = t * NBUF + s

                @pl.when(j >= NBUF)
                def _():
                    scatter_wait(j - NBUF, s)

                fill(j, s)
                scatter_start(j, s)
            return c

        lax.fori_loop(0, ch // NBUF, body, 0)
        for s in range(NBUF):
            scatter_wait(ch - NBUF + s, s)
        plsc.subcore_barrier()
        pltpu.sync_copy(acc_sh.at[sl], out_hbm.at[cid, sl])

    return deg_kernel


@functools.lru_cache(maxsize=None)
def _make_agg(ch, n_p, d, serial_scatter=False):
    npt = n_p // NS
    nv = d // 16
    assert ch % NBUF == 0 and ch >= 2 * NBUF

    @functools.partial(
        pl.kernel,
        mesh=_mesh(),
        out_type=jax.ShapeDtypeStruct((NC, n_p, d), jnp.float32),
        compiler_params=pltpu.CompilerParams(use_tc_tiling_on_sc=False),
        scratch_types=[
            pltpu.VMEM((ch, 128), jnp.int32),
            pltpu.VMEM((ch, 128), jnp.int32),
            pltpu.VMEM((ch, 128), jnp.float32),
            [pltpu.VMEM((128, d), jnp.float32)] * NBUF,
            [pltpu.SemaphoreType.DMA] * NBUF,
            [pltpu.SemaphoreType.DMA] * NBUF,
            pltpu.VMEM_SHARED((n_p, d), jnp.float32),
        ],
    )
    def agg_kernel(table_hbm, src_hbm, dst_hbm, w_hbm, zb_hbm, out_hbm,
                   src_v, dst_v, w_v, rows, gsem, ssem, acc_sh):
        cid = lax.axis_index("c")
        sid = lax.axis_index("s")
        wid = cid * NS + sid
        sl = pl.ds(sid * npt, npt)
        pltpu.sync_copy(zb_hbm, rows[0])
        for b in range(npt // 128):
            pltpu.sync_copy(rows[0],
                            acc_sh.at[pl.ds(sid * npt + b * 128, 128)])
        pltpu.sync_copy(src_hbm.at[wid], src_v)
        pltpu.sync_copy(dst_hbm.at[wid], dst_v)
        pltpu.sync_copy(w_hbm.at[wid], w_v)
        plsc.subcore_barrier()

        def gather_start(j, s):
            pltpu.async_copy(table_hbm.at[src_v.at[j]], rows[s], gsem[s])

        def gather_wait(j, s):
            pltpu.make_async_copy(table_hbm.at[src_v.at[j]], rows[s],
                                  gsem[s]).wait()

        def scale(j, s):
            rv = rows[s]
            for g in range(8):
                wv = w_v[j, pl.ds(g * 16, 16)]
                base = g * 16
                for i in range(16):
                    ws = jnp.take_along_axis(
                        wv, jnp.full((16,), i, jnp.int32), axis=0)
                    for k in range(nv):
                        s2 = pl.ds(k * 16, 16)
                        rv[base + i, s2] = rv[base + i, s2] * ws

        def scatter_start(j, s):
            pltpu.async_copy(rows[s], acc_sh.at[dst_v.at[j]], ssem[s],
                             add=True)

        def scatter_wait(j, s):
            pltpu.make_async_copy(rows[s], acc_sh.at[dst_v.at[j]],
                                  ssem[s]).wait()

        for s in range(NBUF):
            gather_start(s, s)

        if serial_scatter:
            def body(t, c):
                j0 = t * NBUF
                for s in range(NBUF):
                    j = j0 + s
                    sp = (s - 1) % NBUF
                    gather_wait(j, s)
                    scale(j, s)

                    @pl.when(j > 0)
                    def _():
                        scatter_wait(j - 1, sp)

                        @pl.when(j - 1 + NBUF < ch)
                        def _():
                            gather_start(j - 1 + NBUF, sp)

                    scatter_start(j, s)
                return c

            lax.fori_loop(0, ch // NBUF, body, 0)
            scatter_wait(ch - 1, NBUF - 1)
        else:
            def body(t, c):
                j0 = t * NBUF
                for s in range(NBUF):
                    j = j0 + s
                    gather_wait(j, s)
                    scale(j, s)
                    scatter_start(j, s)
                    if s >= 1:
                        jp = j - 1
                        scatter_wait(jp, s - 1)

                        @pl.when(jp + NBUF < ch)
                        def _():
                            gather_start(jp + NBUF, s - 1)

                jl = j0 + NBUF - 1
                scatter_wait(jl, NBUF - 1)

                @pl.when(jl + NBUF < ch)
                def _():
                    gather_start(jl + NBUF, NBUF - 1)

                return c

            lax.fori_loop(0, ch // NBUF, body, 0)
        plsc.subcore_barrier()
        pltpu.sync_copy(acc_sh.at[sl], out_hbm.at[cid, sl])

    return agg_kernel


def _tc1_body(x_ref, w1_ref, dsum_ref, out_ref):
    dinv = lax.rsqrt(dsum_ref[...])
    h = jnp.dot(x_ref[...], w1_ref[...], preferred_element_type=jnp.float32)
    out_ref[...] = h * dinv


def _tc2_body(acc_ref, h1s_ref, dsum_ref, b1_ref, w2_ref, out_ref):
    dinv = lax.rsqrt(dsum_ref[...])
    a = acc_ref[...]
    pre = (a[0] + a[1] + h1s_ref[...]) * dinv + b1_ref[...]
    h2 = jnp.where(pre > 0, pre, 0.01 * pre)
    out_ref[...] = jnp.dot(h2, w2_ref[...],
                           preferred_element_type=jnp.float32) * dinv


def _tc3_body(acc_ref, gs_ref, dsum_ref, b2_ref, out_ref):
    dinv = lax.rsqrt(dsum_ref[...])
    a = acc_ref[...]
    logits = (a[0] + a[1] + gs_ref[...]) * dinv + b2_ref[...]
    col = lax.broadcasted_iota(jnp.int32, logits.shape, 1)
    logits = jnp.where(col < 4, logits, -1e30)
    m = jnp.max(logits, axis=1, keepdims=True)
    ex = jnp.exp(logits - m)
    out_ref[...] = ex / jnp.sum(ex, axis=1, keepdims=True)


def kernel(x_embeddings, edge_index, weights, W1, b1, W2, b2):
    n, d_in = x_embeddings.shape
    e = edge_index.shape[1]
    d_h = W1.shape[1]
    d_out = W2.shape[1]

    n_p = ((n + 1023) // 1024) * 1024
    ch = (e + NW * 128 - 1) // (NW * 128)
    ch = ((ch + NBUF - 1) // NBUF) * NBUF
    e_p = NW * ch * 128
    pad = e_p - e
    d_out_p = 16

    f32 = jnp.float32
    src = edge_index[0]
    dst = edge_index[1]
    pad_idx = (jnp.arange(pad, dtype=jnp.int32) * 97) % n
    src_w = jnp.concatenate([src, pad_idx]).reshape(NW, ch, 128)
    dst_w = jnp.concatenate([dst, pad_idx]).reshape(NW, ch, 128)
    w_flat = jnp.concatenate([weights.astype(f32), jnp.zeros((pad,), f32)])
    w_w = w_flat.reshape(NW, ch, 128)

    x_p = jnp.pad(x_embeddings.astype(f32), ((0, n_p - n), (0, 0)))
    w2_p = jnp.pad(W2.astype(f32), ((0, 0), (0, d_out_p - d_out)))
    b1_r = b1.astype(f32).reshape(1, d_h)
    b2_r = jnp.pad(b2.astype(f32), (0, d_out_p - d_out)).reshape(1, d_out_p)

    zb_h = jnp.zeros((128, d_h), f32)
    zb_o = jnp.zeros((128, d_out_p), f32)

    deg = _make_deg(ch, n_p)(dst_w, w_w, zb_o)
    dsum = deg[0, :, :1] + deg[1, :, :1] + 1.0

    r = 1024
    grid = (n_p // r,)

    h1s = pl.pallas_call(
        _tc1_body,
        grid=grid,
        in_specs=[
            pl.BlockSpec((r, d_in), lambda i: (i, 0)),
            pl.BlockSpec((d_in, d_h), lambda i: (0, 0)),
            pl.BlockSpec((r, 1), lambda i: (i, 0)),
        ],
        out_specs=pl.BlockSpec((r, d_h), lambda i: (i, 0)),
        out_shape=jax.ShapeDtypeStruct((n_p, d_h), f32),
    )(x_p, W1.astype(f32), dsum)

    acc1 = _make_agg(ch, n_p, d_h, True)(h1s, src_w, dst_w, w_w, zb_h)

    gs = pl.pallas_call(
        _tc2_body,
        grid=grid,
        in_specs=[
            pl.BlockSpec((NC, r, d_h), lambda i: (0, i, 0)),
            pl.BlockSpec((r, d_h), lambda i: (i, 0)),
            pl.BlockSpec((r, 1), lambda i: (i, 0)),
            pl.BlockSpec((1, d_h), lambda i: (0, 0)),
            pl.BlockSpec((d_h, d_out_p), lambda i: (0, 0)),
        ],
        out_specs=pl.BlockSpec((r, d_out_p), lambda i: (i, 0)),
        out_shape=jax.ShapeDtypeStruct((n_p, d_out_p), f32),
    )(acc1, h1s, dsum, b1_r, w2_p)

    acc2 = _make_agg(ch, n_p, d_out_p)(gs, src_w, dst_w, w_w, zb_o)

    out = pl.pallas_call(
        _tc3_body,
        grid=grid,
        in_specs=[
            pl.BlockSpec((NC, r, d_out_p), lambda i: (0, i, 0)),
            pl.BlockSpec((r, d_out_p), lambda i: (i, 0)),
            pl.BlockSpec((r, 1), lambda i: (i, 0)),
            pl.BlockSpec((1, d_out_p), lambda i: (0, 0)),
        ],
        out_specs=pl.BlockSpec((r, d_out_p), lambda i: (i, 0)),
        out_shape=jax.ShapeDtypeStruct((n_p, d_out_p), f32),
    )(acc2, gs, dsum, b2_r)

    return out[:n, :d_out]


# speedup vs baseline: 1.0464x; 1.0092x over previous
import functools

import jax
import jax.numpy as jnp
from jax import lax
from jax.experimental import pallas as pl
from jax.experimental.pallas import tpu as pltpu
from jax.experimental.pallas import tpu_sc as plsc

NC = 2
NS = 16
NW = NC * NS


def _mesh():
    return plsc.VectorSubcoreMesh(core_axis_name="c", subcore_axis_name="s")


NBUF = 4


@functools.lru_cache(maxsize=None)
def _make_deg(ch, n_p):
    npt = n_p // NS
    d = 16

    @functools.partial(
        pl.kernel,
        mesh=_mesh(),
        out_type=jax.ShapeDtypeStruct((NC, n_p, d), jnp.float32),
        compiler_params=pltpu.CompilerParams(use_tc_tiling_on_sc=False, needs_layout_passes=False),
        scratch_types=[
            pltpu.VMEM((ch, 128), jnp.int32),
            pltpu.VMEM((ch, 128), jnp.float32),
            [pltpu.VMEM((128, d), jnp.float32)] * NBUF,
            [pltpu.SemaphoreType.DMA] * NBUF,
            pltpu.VMEM_SHARED((n_p, d), jnp.float32),
        ],
    )
    def deg_kernel(dst_hbm, w_hbm, zb_hbm, out_hbm, dst_v, w_v, rows, ssem,
                   acc_sh):
        cid = lax.axis_index("c")
        sid = lax.axis_index("s")
        wid = cid * NS + sid
        sl = pl.ds(sid * npt, npt)
        for s in range(NBUF):
            pltpu.sync_copy(zb_hbm, rows[s])
        for b in range(npt // 128):
            pltpu.sync_copy(rows[0],
                            acc_sh.at[pl.ds(sid * npt + b * 128, 128)])
        pltpu.sync_copy(dst_hbm.at[wid], dst_v)
        pltpu.sync_copy(w_hbm.at[wid], w_v)
        plsc.subcore_barrier()

        def fill(j, s):
            rv = rows[s]
            for g in range(8):
                wv = w_v[j, pl.ds(g * 16, 16)]
                for i in range(16):
                    ws = jnp.take_along_axis(
                        wv, jnp.full((16,), i, jnp.int32), axis=0)
                    sl16 = pl.ds(0, 16)
                    old = rv[g * 16 + i, sl16]
                    rv[g * 16 + i, sl16] = old - old + ws

        def scatter_start(j, s):
            pltpu.async_copy(rows[s], acc_sh.at[dst_v.at[j]], ssem[s],
                             add=True)

        def scatter_wait(j, s):
            pltpu.make_async_copy(rows[s], acc_sh.at[dst_v.at[j]],
                                  ssem[s]).wait()

        def body(t, c):
            for s in range(NBUF):
                j = t * NBUF + s

                @pl.when(j >= NBUF)
                def _():
                    scatter_wait(j - NBUF, s)

                fill(j, s)
                scatter_start(j, s)
            return c

        lax.fori_loop(0, ch // NBUF, body, 0)
        for s in range(NBUF):
            scatter_wait(ch - NBUF + s, s)
        plsc.subcore_barrier()
        pltpu.sync_copy(acc_sh.at[sl], out_hbm.at[cid, sl])

    return deg_kernel


@functools.lru_cache(maxsize=None)
def _make_agg(ch, n_p, d, serial_scatter=False):
    npt = n_p // NS
    nv = d // 16
    assert ch % NBUF == 0 and ch >= 2 * NBUF

    @functools.partial(
        pl.kernel,
        mesh=_mesh(),
        out_type=jax.ShapeDtypeStruct((NC, n_p, d), jnp.float32),
        compiler_params=pltpu.CompilerParams(use_tc_tiling_on_sc=False, needs_layout_passes=False),
        scratch_types=[
            pltpu.VMEM((ch, 128), jnp.int32),
            pltpu.VMEM((ch, 128), jnp.int32),
            pltpu.VMEM((ch, 128), jnp.float32),
            pltpu.VMEM((n_p,), jnp.float32),
            [pltpu.VMEM((128, d), jnp.float32)] * NBUF,
            [pltpu.SemaphoreType.DMA] * NBUF,
            [pltpu.SemaphoreType.DMA] * NBUF,
            pltpu.VMEM_SHARED((n_p, d), jnp.float32),
        ],
    )
    def agg_kernel(table_hbm, src_hbm, dst_hbm, w_hbm, dinv_hbm, zb_hbm,
                   out_hbm, src_v, dst_v, w_v, dinv_v, rows, gsem, ssem,
                   acc_sh):
        cid = lax.axis_index("c")
        sid = lax.axis_index("s")
        wid = cid * NS + sid
        sl = pl.ds(sid * npt, npt)
        pltpu.sync_copy(zb_hbm, rows[0])
        for b in range(npt // 128):
            pltpu.sync_copy(rows[0],
                            acc_sh.at[pl.ds(sid * npt + b * 128, 128)])
        pltpu.sync_copy(src_hbm.at[wid], src_v)
        pltpu.sync_copy(dst_hbm.at[wid], dst_v)
        pltpu.sync_copy(w_hbm.at[wid], w_v)
        pltpu.sync_copy(dinv_hbm, dinv_v)
        plsc.subcore_barrier()

        def gather_start(j, s):
            pltpu.async_copy(table_hbm.at[src_v.at[j]], rows[s], gsem[s])

        def gather_wait(j, s):
            pltpu.make_async_copy(table_hbm.at[src_v.at[j]], rows[s],
                                  gsem[s]).wait()

        def scale(j, s):
            rv = rows[s]
            for g in range(8):
                g16 = pl.ds(g * 16, 16)
                wv = (w_v[j, g16]
                      * plsc.load_gather(dinv_v, [src_v[j, g16]])
                      * plsc.load_gather(dinv_v, [dst_v[j, g16]]))
                base = g * 16
                for i in range(16):
                    ws = jnp.take_along_axis(
                        wv, jnp.full((16,), i, jnp.int32), axis=0)
                    for k in range(nv):
                        s2 = pl.ds(k * 16, 16)
                        rv[base + i, s2] = rv[base + i, s2] * ws

        def scatter_start(j, s):
            pltpu.async_copy(rows[s], acc_sh.at[dst_v.at[j]], ssem[s],
                             add=True)

        def scatter_wait(j, s):
            pltpu.make_async_copy(rows[s], acc_sh.at[dst_v.at[j]],
                                  ssem[s]).wait()

        for s in range(NBUF):
            gather_start(s, s)

        if serial_scatter:
            def body(t, c):
                j0 = t * NBUF
                for s in range(NBUF):
                    j = j0 + s
                    sp = (s - 1) % NBUF
                    gather_wait(j, s)
                    scale(j, s)

                    @pl.when(j > 0)
                    def _():
                        scatter_wait(j - 1, sp)

                        @pl.when(j - 1 + NBUF < ch)
                        def _():
                            gather_start(j - 1 + NBUF, sp)

                    scatter_start(j, s)
                return c

            lax.fori_loop(0, ch // NBUF, body, 0)
            scatter_wait(ch - 1, NBUF - 1)
        else:
            def body(t, c):
                j0 = t * NBUF
                for s in range(NBUF):
                    j = j0 + s
                    gather_wait(j, s)
                    scale(j, s)
                    scatter_start(j, s)
                    if s >= 1:
                        jp = j - 1
                        scatter_wait(jp, s - 1)

                        @pl.when(jp + NBUF < ch)
                        def _():
                            gather_start(jp + NBUF, s - 1)

                jl = j0 + NBUF - 1
                scatter_wait(jl, NBUF - 1)

                @pl.when(jl + NBUF < ch)
                def _():
                    gather_start(jl + NBUF, NBUF - 1)

                return c

            lax.fori_loop(0, ch // NBUF, body, 0)
        plsc.subcore_barrier()
        pltpu.sync_copy(acc_sh.at[sl], out_hbm.at[cid, sl])

    return agg_kernel


def _tc1_body(x_ref, w1_ref, out_ref):
    out_ref[...] = jnp.dot(x_ref[...], w1_ref[...],
                           preferred_element_type=jnp.float32)


def _tc2_body(acc_ref, h1_ref, dsum_ref, b1_ref, w2_ref, out_ref):
    d2 = 1.0 / dsum_ref[...]
    a = acc_ref[...]
    pre = a[0] + a[1] + d2 * h1_ref[...] + b1_ref[...]
    h2 = jnp.where(pre > 0, pre, 0.01 * pre)
    out_ref[...] = jnp.dot(h2, w2_ref[...],
                           preferred_element_type=jnp.float32)


def _tc3_body(acc_ref, gs_ref, dsum_ref, b2_ref, out_ref):
    d2 = 1.0 / dsum_ref[...]
    a = acc_ref[...]
    logits = a[0] + a[1] + d2 * gs_ref[...] + b2_ref[...]
    col = lax.broadcasted_iota(jnp.int32, logits.shape, 1)
    logits = jnp.where(col < 4, logits, -1e30)
    m = jnp.max(logits, axis=1, keepdims=True)
    ex = jnp.exp(logits - m)
    out_ref[...] = ex / jnp.sum(ex, axis=1, keepdims=True)


def kernel(x_embeddings, edge_index, weights, W1, b1, W2, b2):
    n, d_in = x_embeddings.shape
    e = edge_index.shape[1]
    d_h = W1.shape[1]
    d_out = W2.shape[1]

    n_p = ((n + 1023) // 1024) * 1024
    ch = (e + NW * 128 - 1) // (NW * 128)
    ch = ((ch + NBUF - 1) // NBUF) * NBUF
    e_p = NW * ch * 128
    pad = e_p - e
    d_out_p = 16

    f32 = jnp.float32
    src = edge_index[0]
    dst = edge_index[1]
    pad_idx = (jnp.arange(pad, dtype=jnp.int32) * 97) % n
    src_w = jnp.concatenate([src, pad_idx]).reshape(NW, ch, 128)
    dst_w = jnp.concatenate([dst, pad_idx]).reshape(NW, ch, 128)
    w_flat = jnp.concatenate([weights.astype(f32), jnp.zeros((pad,), f32)])
    w_w = w_flat.reshape(NW, ch, 128)

    x_p = jnp.pad(x_embeddings.astype(f32), ((0, n_p - n), (0, 0)))
    w2_p = jnp.pad(W2.astype(f32), ((0, 0), (0, d_out_p - d_out)))
    b1_r = b1.astype(f32).reshape(1, d_h)
    b2_r = jnp.pad(b2.astype(f32), (0, d_out_p - d_out)).reshape(1, d_out_p)

    zb_h = jnp.zeros((128, d_h), f32)
    zb_o = jnp.zeros((128, d_out_p), f32)

    deg = _make_deg(ch, n_p)(dst_w, w_w, zb_o)
    dsum = deg[0, :, :1] + deg[1, :, :1] + 1.0
    dinv = lax.rsqrt(dsum[:, 0])

    r = 1024
    grid = (n_p // r,)

    h1 = pl.pallas_call(
        _tc1_body,
        grid=grid,
        in_specs=[
            pl.BlockSpec((r, d_in), lambda i: (i, 0)),
            pl.BlockSpec((d_in, d_h), lambda i: (0, 0)),
        ],
        out_specs=pl.BlockSpec((r, d_h), lambda i: (i, 0)),
        out_shape=jax.ShapeDtypeStruct((n_p, d_h), f32),
    )(x_p, W1.astype(f32))

    acc1 = _make_agg(ch, n_p, d_h, True)(h1, src_w, dst_w, w_w, dinv, zb_h)

    gs = pl.pallas_call(
        _tc2_body,
        grid=grid,
        in_specs=[
            pl.BlockSpec((NC, r, d_h), lambda i: (0, i, 0)),
            pl.BlockSpec((r, d_h), lambda i: (i, 0)),
            pl.BlockSpec((r, 1), lambda i: (i, 0)),
            pl.BlockSpec((1, d_h), lambda i: (0, 0)),
            pl.BlockSpec((d_h, d_out_p), lambda i: (0, 0)),
        ],
        out_specs=pl.BlockSpec((r, d_out_p), lambda i: (i, 0)),
        out_shape=jax.ShapeDtypeStruct((n_p, d_out_p), f32),
    )(acc1, h1, dsum, b1_r, w2_p)

    acc2 = _make_agg(ch, n_p, d_out_p)(gs, src_w, dst_w, w_w, dinv, zb_o)

    out = pl.pallas_call(
        _tc3_body,
        grid=grid,
        in_specs=[
            pl.BlockSpec((NC, r, d_out_p), lambda i: (0, i, 0)),
            pl.BlockSpec((r, d_out_p), lambda i: (i, 0)),
            pl.BlockSpec((r, 1), lambda i: (i, 0)),
            pl.BlockSpec((1, d_out_p), lambda i: (0, 0)),
        ],
        out_specs=pl.BlockSpec((r, d_out_p), lambda i: (i, 0)),
        out_shape=jax.ShapeDtypeStruct((n_p, d_out_p), f32),
    )(acc2, gs, dsum, b2_r)

    return out[:n, :d_out]
